# Initial kernel scaffold; baseline (speedup 1.0000x reference)
import functools

import jax
import jax.numpy as jnp
from jax import lax
from jax.experimental import pallas as pl
from jax.experimental.pallas import tpu as pltpu
from jax.experimental.pallas import tpu_sc as plsc

N_ATOMS = 100000
N_EDGES = 6400000
CUTOFF = 5.0
CUTON = 3.5
INV_RANGE = 1.0 / (CUTOFF - CUTON)

NC = 2
NS = 16
NW = NC * NS
EPT = N_EDGES // NW
C = 2000
ZTAB = 96


def _sc_body(atoms_h, dist_h, ii_h, jj_h, params_h, logz_h, out_h,
             atab, pv, lv, za, ii, jj, dd, rep, erep):
    c = lax.axis_index("c")
    s = lax.axis_index("s")
    wid = s * NC + c

    pltpu.sync_copy(atoms_h, atab)
    pltpu.sync_copy(params_h, pv)
    pltpu.sync_copy(logz_h, lv)

    s_aexp = pv[0]
    inv_a = pv[1]
    c0 = pv[2]
    c1 = pv[3]
    c2 = pv[4]
    c3 = pv[5]
    e0 = pv[6]
    e1 = pv[7]
    e2 = pv[8]
    e3 = pv[9]

    for k in range(ZTAB // 16):
        za[pl.ds(16 * k, 16)] = jnp.exp(s_aexp * lv[pl.ds(16 * k, 16)])

    zeros16 = jnp.zeros((16,), jnp.float32)

    def _zfill(v, carry):
        rep[pl.ds(16 * v, 16)] = zeros16
        return carry

    lax.fori_loop(0, C // 16, _zfill, 0)

    @pl.when(s == 0)
    def _zero_erep():
        def _zc(k, carry):
            pltpu.sync_copy(rep, erep.at[pl.ds(k * C, C)])
            return carry
        lax.fori_loop(0, N_ATOMS // C, _zc, 0)

    plsc.subcore_barrier()

    base0 = wid * EPT

    def _chunk(cidx, carry):
        base = pl.multiple_of(base0 + cidx * C, C)
        pltpu.sync_copy(ii_h.at[pl.ds(base, C)], ii)
        pltpu.sync_copy(jj_h.at[pl.ds(base, C)], jj)
        pltpu.sync_copy(dist_h.at[pl.ds(base, C)], dd)

        def _vec(v, vcarry):
            o = 16 * v
            iiv = ii[pl.ds(o, 16)]
            jjv = jj[pl.ds(o, 16)]
            d = dd[pl.ds(o, 16)]
            ani = plsc.load_gather(atab, [iiv])
            anj = plsc.load_gather(atab, [jjv])
            zi = plsc.load_gather(za, [ani])
            zj = plsc.load_gather(za, [anj])
            arg = d * ((zi + zj) * inv_a)
            phi = (c0 * jnp.exp(-e0 * arg) + c1 * jnp.exp(-e1 * arg)
                   + c2 * jnp.exp(-e2 * arg) + c3 * jnp.exp(-e3 * arg))
            x = (CUTOFF - d) * INV_RANGE
            poly = ((6.0 * x - 15.0) * x + 10.0) * x * x * x
            sw = jnp.where(d < CUTON, jnp.ones_like(d),
                           jnp.where(d >= CUTOFF, jnp.zeros_like(d), poly))
            anif = ani.astype(jnp.float32)
            anjf = anj.astype(jnp.float32)
            r = 0.5 * anif * anjf / d * phi * sw
            rep[pl.ds(o, 16)] = r
            return vcarry

        lax.fori_loop(0, C // 16, _vec, 0)
        pltpu.sync_copy(rep, erep.at[ii], add=True)
        return carry

    lax.fori_loop(0, EPT // C, _chunk, 0)

    plsc.subcore_barrier()

    @pl.when(s == 0)
    def _writeback():
        pltpu.sync_copy(erep, out_h.at[c])


_sc_kernel = functools.partial(
    pl.kernel,
    mesh=plsc.VectorSubcoreMesh(core_axis_name="c", subcore_axis_name="s"),
    out_type=jax.ShapeDtypeStruct((NC, N_ATOMS), jnp.float32),
    scratch_types=[
        pltpu.VMEM((N_ATOMS,), jnp.int32),
        pltpu.VMEM((16,), jnp.float32),
        pltpu.VMEM((ZTAB,), jnp.float32),
        pltpu.VMEM((ZTAB,), jnp.float32),
        pltpu.VMEM((C,), jnp.int32),
        pltpu.VMEM((C,), jnp.int32),
        pltpu.VMEM((C,), jnp.float32),
        pltpu.VMEM((C,), jnp.float32),
        pltpu.VMEM_SHARED((N_ATOMS,), jnp.float32),
    ],
)(_sc_body)


def _add_body(p_ref, o_ref):
    o_ref[...] = p_ref[0, :] + p_ref[1, :]


def _combine(partials):
    return pl.pallas_call(
        _add_body,
        out_shape=jax.ShapeDtypeStruct((N_ATOMS,), jnp.float32),
    )(partials)


def kernel(atomic_numbers, distances, idx_i, idx_j, a_coefficient,
           a_exponent, phi_coefficients, phi_exponents):
    abs_c = jnp.abs(phi_coefficients)
    coeffs = abs_c / jnp.maximum(jnp.sum(abs_c), 1e-12)
    exps = jnp.abs(phi_exponents)
    s_aexp = jnp.abs(a_exponent)
    inv_a = 1.0 / jnp.abs(a_coefficient)
    params = jnp.zeros((16,), jnp.float32)
    params = params.at[0].set(s_aexp[0]).at[1].set(inv_a[0])
    params = params.at[2:6].set(coeffs).at[6:10].set(exps)
    logz = jnp.log(jnp.maximum(jnp.arange(ZTAB, dtype=jnp.float32), 1.0))
    partials = _sc_kernel(atomic_numbers, distances, idx_i, idx_j,
                          params, logz)
    return _combine(partials)


# speedup vs baseline: 358.7974x; 358.7974x over previous
import functools

import jax
import jax.numpy as jnp
from jax import lax
from jax.experimental import pallas as pl
from jax.experimental.pallas import tpu as pltpu
from jax.experimental.pallas import tpu_sc as plsc

N_ATOMS = 100000
N_EDGES = 6400000
CUTOFF = 5.0
CUTON = 3.5
INV_RANGE = 1.0 / (CUTOFF - CUTON)

NC = 2
NS = 16
NW = NC * NS
EPT = N_EDGES // NW
C = 2000
ZTAB = 96


def _sc_body(atoms_h, dist_h, ii_h, jj_h, params_h, logz_h, out_h,
             atab, pv, lv, za, ii, jj, dd, rep, erep):
    c = lax.axis_index("c")
    s = lax.axis_index("s")
    wid = s * NC + c

    pltpu.sync_copy(atoms_h, atab)
    pltpu.sync_copy(params_h, pv)
    pltpu.sync_copy(logz_h, lv)

    pvv = pv[...]
    s_aexp = pvv[0]
    inv_a = pvv[1]
    c0 = pvv[2]
    c1 = pvv[3]
    c2 = pvv[4]
    c3 = pvv[5]
    e0 = pvv[6]
    e1 = pvv[7]
    e2 = pvv[8]
    e3 = pvv[9]

    for k in range(ZTAB // 16):
        za[pl.ds(16 * k, 16)] = jnp.exp(s_aexp * lv[pl.ds(16 * k, 16)])

    zeros16 = jnp.zeros((16,), jnp.float32)

    def _zfill(v, carry):
        rep[pl.ds(16 * v, 16)] = zeros16
        return carry

    lax.fori_loop(0, C // 16, _zfill, 0)

    @pl.when(s == 0)
    def _zero_erep():
        def _zc(k, carry):
            pltpu.sync_copy(rep, erep.at[pl.ds(k * C, C)])
            return carry
        lax.fori_loop(0, N_ATOMS // C, _zc, 0)

    plsc.subcore_barrier()

    base0 = wid * EPT

    def _chunk(cidx, carry):
        base = pl.multiple_of(base0 + cidx * C, C)
        pltpu.sync_copy(ii_h.at[pl.ds(base, C)], ii)
        pltpu.sync_copy(jj_h.at[pl.ds(base, C)], jj)
        pltpu.sync_copy(dist_h.at[pl.ds(base, C)], dd)

        def _vec(v, vcarry):
            o = 16 * v
            iiv = ii[pl.ds(o, 16)]
            jjv = jj[pl.ds(o, 16)]
            d = dd[pl.ds(o, 16)]
            ani = plsc.load_gather(atab, [iiv])
            anj = plsc.load_gather(atab, [jjv])
            zi = plsc.load_gather(za, [ani])
            zj = plsc.load_gather(za, [anj])
            arg = d * ((zi + zj) * inv_a)
            phi = (c0 * jnp.exp(-e0 * arg) + c1 * jnp.exp(-e1 * arg)
                   + c2 * jnp.exp(-e2 * arg) + c3 * jnp.exp(-e3 * arg))
            x = (CUTOFF - d) * INV_RANGE
            poly = ((6.0 * x - 15.0) * x + 10.0) * x * x * x
            sw = jnp.where(d < CUTON, jnp.ones_like(d),
                           jnp.where(d >= CUTOFF, jnp.zeros_like(d), poly))
            anif = ani.astype(jnp.float32)
            anjf = anj.astype(jnp.float32)
            r = 0.5 * anif * anjf / d * phi * sw
            rep[pl.ds(o, 16)] = r
            return vcarry

        lax.fori_loop(0, C // 16, _vec, 0)
        pltpu.sync_copy(rep, erep.at[ii], add=True)
        return carry

    lax.fori_loop(0, EPT // C, _chunk, 0)

    plsc.subcore_barrier()

    @pl.when(s == 0)
    def _writeback():
        pltpu.sync_copy(erep, out_h.at[c])


_sc_kernel = functools.partial(
    pl.kernel,
    mesh=plsc.VectorSubcoreMesh(core_axis_name="c", subcore_axis_name="s"),
    out_type=jax.ShapeDtypeStruct((NC, N_ATOMS), jnp.float32),
    scratch_types=[
        pltpu.VMEM((N_ATOMS,), jnp.int32),
        pltpu.VMEM((16,), jnp.float32),
        pltpu.VMEM((ZTAB,), jnp.float32),
        pltpu.VMEM((ZTAB,), jnp.float32),
        pltpu.VMEM((C,), jnp.int32),
        pltpu.VMEM((C,), jnp.int32),
        pltpu.VMEM((C,), jnp.float32),
        pltpu.VMEM((C,), jnp.float32),
        pltpu.VMEM_SHARED((N_ATOMS,), jnp.float32),
    ],
    compiler_params=pltpu.CompilerParams(needs_layout_passes=False),
)(_sc_body)


def _add_body(p_ref, o_ref):
    o_ref[...] = p_ref[0, :] + p_ref[1, :]


def _combine(partials):
    return pl.pallas_call(
        _add_body,
        out_shape=jax.ShapeDtypeStruct((N_ATOMS,), jnp.float32),
    )(partials)


def kernel(atomic_numbers, distances, idx_i, idx_j, a_coefficient,
           a_exponent, phi_coefficients, phi_exponents):
    abs_c = jnp.abs(phi_coefficients)
    coeffs = abs_c / jnp.maximum(jnp.sum(abs_c), 1e-12)
    exps = jnp.abs(phi_exponents)
    s_aexp = jnp.abs(a_exponent)
    inv_a = 1.0 / jnp.abs(a_coefficient)
    params = jnp.zeros((16,), jnp.float32)
    params = params.at[0].set(s_aexp[0]).at[1].set(inv_a[0])
    params = params.at[2:6].set(coeffs).at[6:10].set(exps)
    logz = jnp.log(jnp.maximum(jnp.arange(ZTAB, dtype=jnp.float32), 1.0))
    partials = _sc_kernel(atomic_numbers, distances, idx_i, idx_j,
                          params, logz)
    return _combine(partials)


# speedup vs baseline: 596.5548x; 1.6627x over previous
import functools

import jax
import jax.numpy as jnp
from jax import lax
from jax.experimental import pallas as pl
from jax.experimental.pallas import tpu as pltpu
from jax.experimental.pallas import tpu_sc as plsc

N_ATOMS = 100000
N_EDGES = 6400000
CUTOFF = 5.0
CUTON = 3.5
INV_RANGE = 1.0 / (CUTOFF - CUTON)

NC = 2
NS = 16
NW = NC * NS
EPT = N_EDGES // NW
C = 2000
G = EPT // C
ZTAB = 96


def _sc_body(atoms_h, dist_h, ii_h, jj_h, params_h, logz_h, out_h,
             atab, pv, lv, za,
             ii0, ii1, jj0, jj1, dd0, dd1, rep0, rep1, sii0, sii1,
             in_sem0, in_sem1, sc_sem0, sc_sem1, erep):
    c = lax.axis_index("c")
    s = lax.axis_index("s")
    wid = s * NC + c
    base0 = wid * EPT

    iib = (ii0, ii1)
    jjb = (jj0, jj1)
    ddb = (dd0, dd1)
    repb = (rep0, rep1)
    siib = (sii0, sii1)
    in_sems = (in_sem0, in_sem1)
    sc_sems = (sc_sem0, sc_sem1)

    def in_copies(g, b):
        base = pl.multiple_of(base0 + g * C, C)
        return (
            pltpu.make_async_copy(ii_h.at[pl.ds(base, C)], iib[b], in_sems[b]),
            pltpu.make_async_copy(jj_h.at[pl.ds(base, C)], jjb[b], in_sems[b]),
            pltpu.make_async_copy(dist_h.at[pl.ds(base, C)], ddb[b], in_sems[b]),
        )

    for b in range(2):
        for cp in in_copies(b, b):
            cp.start()

    pltpu.sync_copy(atoms_h, atab)
    pltpu.sync_copy(params_h, pv)
    pltpu.sync_copy(logz_h, lv)

    pvv = pv[...]
    s_aexp = pvv[0]
    inv_a = pvv[1]
    c0 = pvv[2]
    c1 = pvv[3]
    c2 = pvv[4]
    c3 = pvv[5]
    e0 = pvv[6]
    e1 = pvv[7]
    e2 = pvv[8]
    e3 = pvv[9]

    for k in range(ZTAB // 16):
        za[pl.ds(16 * k, 16)] = inv_a * jnp.exp(s_aexp * lv[pl.ds(16 * k, 16)])

    zeros16 = jnp.zeros((16,), jnp.float32)

    def _zfill(v, carry):
        rep0[pl.ds(16 * v, 16)] = zeros16
        return carry

    lax.fori_loop(0, C // 16, _zfill, 0)

    @pl.when(s == 0)
    def _zero_erep():
        def _zc(k, carry):
            pltpu.sync_copy(rep0, erep.at[pl.ds(k * C, C)])
            return carry
        lax.fori_loop(0, N_ATOMS // C, _zc, 0)

    plsc.subcore_barrier()

    def _compute_chunk(b):
        iiw, jjw, ddw = iib[b], jjb[b], ddb[b]
        repw, siiw = repb[b], siib[b]

        def _vec(v, vcarry):
            o = 16 * v
            iiv = iiw[pl.ds(o, 16)]
            jjv = jjw[pl.ds(o, 16)]
            d = ddw[pl.ds(o, 16)]
            ani = plsc.load_gather(atab, [iiv])
            anj = plsc.load_gather(atab, [jjv])
            zi = plsc.load_gather(za, [ani])
            zj = plsc.load_gather(za, [anj])
            arg = d * (zi + zj)
            phi = (c0 * jnp.exp(-e0 * arg) + c1 * jnp.exp(-e1 * arg)
                   + c2 * jnp.exp(-e2 * arg) + c3 * jnp.exp(-e3 * arg))
            x = (CUTOFF - d) * INV_RANGE
            poly = ((6.0 * x - 15.0) * x + 10.0) * x * x * x
            sw = jnp.where(d < CUTON, jnp.ones_like(d),
                           jnp.where(d >= CUTOFF, jnp.zeros_like(d), poly))
            anif = ani.astype(jnp.float32)
            anjf = anj.astype(jnp.float32)
            r = anif * anjf / d * phi * sw
            repw[pl.ds(o, 16)] = r
            siiw[pl.ds(o, 16)] = iiv
            return vcarry

        lax.fori_loop(0, C // 16, _vec, 0)

    def _group(grp, carry):
        for b in range(2):
            g = grp * 2 + b
            for cp in in_copies(g, b):
                cp.wait()
            @pl.when(g >= 2)
            def _wait_prev_scatter():
                pltpu.make_async_copy(
                    repb[b], erep.at[siib[b]], sc_sems[b]).wait()
            _compute_chunk(b)
            pltpu.async_copy(repb[b], erep.at[siib[b]], sc_sems[b], add=True)
            @pl.when(g + 2 < G)
            def _prefetch():
                for cp in in_copies(g + 2, b):
                    cp.start()
        return carry

    lax.fori_loop(0, G // 2, _group, 0)

    for b in range(2):
        pltpu.make_async_copy(repb[b], erep.at[siib[b]], sc_sems[b]).wait()

    plsc.subcore_barrier()

    @pl.when(s == 0)
    def _writeback():
        pltpu.sync_copy(erep, out_h.at[c])


_sc_kernel = functools.partial(
    pl.kernel,
    mesh=plsc.VectorSubcoreMesh(core_axis_name="c", subcore_axis_name="s"),
    out_type=jax.ShapeDtypeStruct((NC, N_ATOMS), jnp.float32),
    scratch_types=[
        pltpu.VMEM((N_ATOMS,), jnp.int32),
        pltpu.VMEM((16,), jnp.float32),
        pltpu.VMEM((ZTAB,), jnp.float32),
        pltpu.VMEM((ZTAB,), jnp.float32),
        pltpu.VMEM((C,), jnp.int32),
        pltpu.VMEM((C,), jnp.int32),
        pltpu.VMEM((C,), jnp.int32),
        pltpu.VMEM((C,), jnp.int32),
        pltpu.VMEM((C,), jnp.float32),
        pltpu.VMEM((C,), jnp.float32),
        pltpu.VMEM((C,), jnp.float32),
        pltpu.VMEM((C,), jnp.float32),
        pltpu.VMEM((C,), jnp.int32),
        pltpu.VMEM((C,), jnp.int32),
        pltpu.SemaphoreType.DMA,
        pltpu.SemaphoreType.DMA,
        pltpu.SemaphoreType.DMA,
        pltpu.SemaphoreType.DMA,
        pltpu.VMEM_SHARED((N_ATOMS,), jnp.float32),
    ],
    compiler_params=pltpu.CompilerParams(needs_layout_passes=False),
)(_sc_body)


def _add_body(p_ref, o_ref):
    o_ref[...] = p_ref[0, :] + p_ref[1, :]


def _combine(partials):
    return pl.pallas_call(
        _add_body,
        out_shape=jax.ShapeDtypeStruct((N_ATOMS,), jnp.float32),
    )(partials)


def kernel(atomic_numbers, distances, idx_i, idx_j, a_coefficient,
           a_exponent, phi_coefficients, phi_exponents):
    abs_c = jnp.abs(phi_coefficients)
    coeffs = 0.5 * abs_c / jnp.maximum(jnp.sum(abs_c), 1e-12)
    exps = jnp.abs(phi_exponents)
    s_aexp = jnp.abs(a_exponent)
    inv_a = 1.0 / jnp.abs(a_coefficient)
    params = jnp.zeros((16,), jnp.float32)
    params = params.at[0].set(s_aexp[0]).at[1].set(inv_a[0])
    params = params.at[2:6].set(coeffs).at[6:10].set(exps)
    logz = jnp.log(jnp.maximum(jnp.arange(ZTAB, dtype=jnp.float32), 1.0))
    partials = _sc_kernel(atomic_numbers, distances, idx_i, idx_j,
                          params, logz)
    return _combine(partials)


# speedup vs baseline: 1416.7702x; 2.3749x over previous
import functools

import jax
import jax.numpy as jnp
from jax import lax
from jax.experimental import pallas as pl
from jax.experimental.pallas import tpu as pltpu
from jax.experimental.pallas import tpu_sc as plsc

N_ATOMS = 100000
N_EDGES = 6400000
CUTOFF = 5.0
CUTON = 3.5
INV_RANGE = 1.0 / (CUTOFF - CUTON)

NC = 2
NS = 16
NW = NC * NS
EPT = N_EDGES // NW
C = 2000
G = EPT // C
ZTAB = 96


def _sc_body(atoms_h, dist_h, ii_h, jj_h, params_h, logz_h, out_h,
             atab, pv, lv, za,
             ii0, ii1, jj0, jj1, dd0, dd1, rep0, rep1, sii0, sii1,
             in_sem0, in_sem1, sc_sem0, sc_sem1, erep):
    c = lax.axis_index("c")
    s = lax.axis_index("s")
    wid = s * NC + c
    base0 = wid * EPT

    iib = (ii0, ii1)
    jjb = (jj0, jj1)
    ddb = (dd0, dd1)
    repb = (rep0, rep1)
    siib = (sii0, sii1)
    in_sems = (in_sem0, in_sem1)
    sc_sems = (sc_sem0, sc_sem1)

    def in_copies(g, b):
        base = pl.multiple_of(base0 + g * C, C)
        return (
            pltpu.make_async_copy(ii_h.at[pl.ds(base, C)], iib[b], in_sems[b]),
            pltpu.make_async_copy(jj_h.at[pl.ds(base, C)], jjb[b], in_sems[b]),
            pltpu.make_async_copy(dist_h.at[pl.ds(base, C)], ddb[b], in_sems[b]),
        )

    for b in range(2):
        for cp in in_copies(b, b):
            cp.start()

    pltpu.sync_copy(atoms_h, atab)
    pltpu.sync_copy(params_h, pv)
    pltpu.sync_copy(logz_h, lv)

    pvv = pv[...]
    s_aexp = pvv[0]
    inv_a = pvv[1]
    c0 = pvv[2]
    c1 = pvv[3]
    c2 = pvv[4]
    c3 = pvv[5]
    e0 = pvv[6]
    e1 = pvv[7]
    e2 = pvv[8]
    e3 = pvv[9]

    for k in range(ZTAB // 16):
        za[pl.ds(16 * k, 16)] = inv_a * jnp.exp(s_aexp * lv[pl.ds(16 * k, 16)])

    zeros16 = jnp.zeros((16,), jnp.float32)

    def _zfill(v, carry):
        rep0[pl.ds(16 * v, 16)] = zeros16
        return carry

    lax.fori_loop(0, C // 16, _zfill, 0)

    @pl.when(s == 0)
    def _zero_erep():
        def _zc(k, carry):
            pltpu.sync_copy(rep0, erep.at[pl.ds(k * C, C)])
            return carry
        lax.fori_loop(0, N_ATOMS // C, _zc, 0)

    plsc.subcore_barrier()

    def _compute_chunk(b):
        iiw, jjw, ddw = iib[b], jjb[b], ddb[b]
        repw, siiw = repb[b], siib[b]

        @plsc.parallel_loop(0, C, step=16, unroll=4)
        def _vec(o):
            iiv = iiw[pl.ds(o, 16)]
            jjv = jjw[pl.ds(o, 16)]
            d = ddw[pl.ds(o, 16)]
            ani = plsc.load_gather(atab, [iiv])
            anj = plsc.load_gather(atab, [jjv])
            zi = plsc.load_gather(za, [ani])
            zj = plsc.load_gather(za, [anj])
            arg = d * (zi + zj)
            phi = (c0 * jnp.exp(-e0 * arg) + c1 * jnp.exp(-e1 * arg)
                   + c2 * jnp.exp(-e2 * arg) + c3 * jnp.exp(-e3 * arg))
            x = (CUTOFF - d) * INV_RANGE
            poly = ((6.0 * x - 15.0) * x + 10.0) * x * x * x
            sw = jnp.where(d < CUTON, jnp.ones_like(d),
                           jnp.where(d >= CUTOFF, jnp.zeros_like(d), poly))
            anif = ani.astype(jnp.float32)
            anjf = anj.astype(jnp.float32)
            r = anif * anjf / d * phi * sw
            repw[pl.ds(o, 16)] = r
            siiw[pl.ds(o, 16)] = iiv

    def _group(grp, carry):
        for b in range(2):
            g = grp * 2 + b
            for cp in in_copies(g, b):
                cp.wait()
            @pl.when(g >= 2)
            def _wait_prev_scatter():
                pltpu.make_async_copy(
                    repb[b], erep.at[siib[b]], sc_sems[b]).wait()
            _compute_chunk(b)
            pltpu.async_copy(repb[b], erep.at[siib[b]], sc_sems[b], add=True)
            @pl.when(g + 2 < G)
            def _prefetch():
                for cp in in_copies(g + 2, b):
                    cp.start()
        return carry

    lax.fori_loop(0, G // 2, _group, 0)

    for b in range(2):
        pltpu.make_async_copy(repb[b], erep.at[siib[b]], sc_sems[b]).wait()

    plsc.subcore_barrier()

    @pl.when(s == 0)
    def _writeback():
        pltpu.sync_copy(erep, out_h.at[c])


_sc_kernel = functools.partial(
    pl.kernel,
    mesh=plsc.VectorSubcoreMesh(core_axis_name="c", subcore_axis_name="s"),
    out_type=jax.ShapeDtypeStruct((NC, N_ATOMS), jnp.float32),
    scratch_types=[
        pltpu.VMEM((N_ATOMS,), jnp.int32),
        pltpu.VMEM((16,), jnp.float32),
        pltpu.VMEM((ZTAB,), jnp.float32),
        pltpu.VMEM((ZTAB,), jnp.float32),
        pltpu.VMEM((C,), jnp.int32),
        pltpu.VMEM((C,), jnp.int32),
        pltpu.VMEM((C,), jnp.int32),
        pltpu.VMEM((C,), jnp.int32),
        pltpu.VMEM((C,), jnp.float32),
        pltpu.VMEM((C,), jnp.float32),
        pltpu.VMEM((C,), jnp.float32),
        pltpu.VMEM((C,), jnp.float32),
        pltpu.VMEM((C,), jnp.int32),
        pltpu.VMEM((C,), jnp.int32),
        pltpu.SemaphoreType.DMA,
        pltpu.SemaphoreType.DMA,
        pltpu.SemaphoreType.DMA,
        pltpu.SemaphoreType.DMA,
        pltpu.VMEM_SHARED((N_ATOMS,), jnp.float32),
    ],
    compiler_params=pltpu.CompilerParams(needs_layout_passes=False),
)(_sc_body)


def _add_body(p_ref, o_ref):
    o_ref[...] = p_ref[0, :] + p_ref[1, :]


def _combine(partials):
    return pl.pallas_call(
        _add_body,
        out_shape=jax.ShapeDtypeStruct((N_ATOMS,), jnp.float32),
    )(partials)


def kernel(atomic_numbers, distances, idx_i, idx_j, a_coefficient,
           a_exponent, phi_coefficients, phi_exponents):
    abs_c = jnp.abs(phi_coefficients)
    coeffs = 0.5 * abs_c / jnp.maximum(jnp.sum(abs_c), 1e-12)
    exps = jnp.abs(phi_exponents)
    s_aexp = jnp.abs(a_exponent)
    inv_a = 1.0 / jnp.abs(a_coefficient)
    params = jnp.zeros((16,), jnp.float32)
    params = params.at[0].set(s_aexp[0]).at[1].set(inv_a[0])
    params = params.at[2:6].set(coeffs).at[6:10].set(exps)
    logz = jnp.log(jnp.maximum(jnp.arange(ZTAB, dtype=jnp.float32), 1.0))
    partials = _sc_kernel(atomic_numbers, distances, idx_i, idx_j,
                          params, logz)
    return _combine(partials)
